# baseline (speedup 1.0000x reference)
import jax
import jax.numpy as jnp
from jax import lax
from jax.experimental import pallas as pl
from jax.experimental.pallas import tpu as pltpu

_HB = 8
_KP = 1024


def _tc_body(alpha_ref, t_ref, x_ref, n_ref, out_ref, nout_ref, sa_ref, sn_ref):
    @pl.when(pl.program_id(0) == 0)
    def _():
        tv = t_ref[...]
        kk = lax.broadcasted_iota(jnp.int32, (_KP, 128), 0)
        oh = (kk == jnp.broadcast_to(tv, (_KP, 128))).astype(jnp.float32)
        a = jnp.dot(
            alpha_ref[...],
            oh,
            preferred_element_type=jnp.float32,
            precision=lax.Precision.HIGHEST,
        )
        sa_ref[...] = jnp.sqrt(a)
        sn_ref[...] = jnp.sqrt(1.0 - a)

    sa = sa_ref[...].reshape(1, 1, 1, 128)
    sn = sn_ref[...].reshape(1, 1, 1, 128)
    n = n_ref[...]
    out_ref[...] = sa * x_ref[...] + sn * n
    nout_ref[...] = n


def kernel(x_0, noise, t, alpha_cumprod):
    B, C, H, W = x_0.shape
    T = alpha_cumprod.shape[0]
    xT = jnp.transpose(x_0, (1, 2, 3, 0))
    nT = jnp.transpose(noise, (1, 2, 3, 0))
    alpha_p = jnp.pad(alpha_cumprod, (0, _KP - T)).reshape(1, _KP)
    t2 = t.reshape(1, B)

    blk = (C, _HB, W, B)
    bmap = lambda h: (0, h, 0, 0)
    outT, noutT = pl.pallas_call(
        _tc_body,
        grid=(H // _HB,),
        in_specs=[
            pl.BlockSpec((1, _KP), lambda h: (0, 0)),
            pl.BlockSpec((1, B), lambda h: (0, 0)),
            pl.BlockSpec(blk, bmap),
            pl.BlockSpec(blk, bmap),
        ],
        out_specs=[pl.BlockSpec(blk, bmap), pl.BlockSpec(blk, bmap)],
        out_shape=[
            jax.ShapeDtypeStruct((C, H, W, B), x_0.dtype),
            jax.ShapeDtypeStruct((C, H, W, B), x_0.dtype),
        ],
        scratch_shapes=[
            pltpu.VMEM((1, B), jnp.float32),
            pltpu.VMEM((1, B), jnp.float32),
        ],
    )(alpha_p, t2, xT, nT)
    return (
        jnp.transpose(outT, (3, 0, 1, 2)),
        jnp.transpose(noutT, (3, 0, 1, 2)),
        t,
    )


# speedup vs baseline: 1.0070x; 1.0070x over previous
import jax
import jax.numpy as jnp
from jax import lax
from jax.experimental import pallas as pl
from jax.experimental.pallas import tpu as pltpu

_HB = 28
_KP = 1024


def _tc_body(alpha_ref, t_ref, x_ref, n_ref, out_ref, nout_ref, sa_ref, sn_ref):
    @pl.when(pl.program_id(0) == 0)
    def _():
        tv = t_ref[...]
        kk = lax.broadcasted_iota(jnp.int32, (_KP, 128), 0)
        oh = (kk == jnp.broadcast_to(tv, (_KP, 128))).astype(jnp.float32)
        a = jnp.dot(
            alpha_ref[...],
            oh,
            preferred_element_type=jnp.float32,
            precision=lax.Precision.HIGHEST,
        )
        sa_ref[...] = jnp.sqrt(a)
        sn_ref[...] = jnp.sqrt(1.0 - a)

    sa = sa_ref[...].reshape(1, 1, 1, 128)
    sn = sn_ref[...].reshape(1, 1, 1, 128)
    n = n_ref[...]
    out_ref[...] = sa * x_ref[...] + sn * n
    nout_ref[...] = n


def kernel(x_0, noise, t, alpha_cumprod):
    B, C, H, W = x_0.shape
    T = alpha_cumprod.shape[0]
    xT = jnp.transpose(x_0, (1, 2, 3, 0))
    nT = jnp.transpose(noise, (1, 2, 3, 0))
    alpha_p = jnp.pad(alpha_cumprod, (0, _KP - T)).reshape(1, _KP)
    t2 = t.reshape(1, B)

    blk = (C, _HB, W, B)
    bmap = lambda h: (0, h, 0, 0)
    outT, noutT = pl.pallas_call(
        _tc_body,
        grid=(H // _HB,),
        in_specs=[
            pl.BlockSpec((1, _KP), lambda h: (0, 0)),
            pl.BlockSpec((1, B), lambda h: (0, 0)),
            pl.BlockSpec(blk, bmap),
            pl.BlockSpec(blk, bmap),
        ],
        out_specs=[
            pl.BlockSpec(blk, bmap, pipeline_mode=pl.Buffered(buffer_count=1)),
            pl.BlockSpec(blk, bmap, pipeline_mode=pl.Buffered(buffer_count=1)),
        ],
        out_shape=[
            jax.ShapeDtypeStruct((C, H, W, B), x_0.dtype),
            jax.ShapeDtypeStruct((C, H, W, B), x_0.dtype),
        ],
        scratch_shapes=[
            pltpu.VMEM((1, B), jnp.float32),
            pltpu.VMEM((1, B), jnp.float32),
        ],
    )(alpha_p, t2, xT, nT)
    return (
        jnp.transpose(outT, (3, 0, 1, 2)),
        jnp.transpose(noutT, (3, 0, 1, 2)),
        t,
    )


# speedup vs baseline: 1.0178x; 1.0108x over previous
import jax
import jax.numpy as jnp
from jax import lax
from jax.experimental import pallas as pl
from jax.experimental.pallas import tpu as pltpu

_RB = 56
_KP = 1024


def _tc_body(alpha_ref, t_ref, x_ref, n_ref, out_ref, nout_ref, sa_ref, sn_ref):
    @pl.when(pl.program_id(0) == 0)
    def _():
        tv = t_ref[...]
        kk = lax.broadcasted_iota(jnp.int32, (_KP, 128), 0)
        oh = (kk == jnp.broadcast_to(tv, (_KP, 128))).astype(jnp.float32)
        a = jnp.dot(
            alpha_ref[...],
            oh,
            preferred_element_type=jnp.float32,
            precision=lax.Precision.HIGHEST,
        )
        sa_ref[...] = jnp.sqrt(a)
        sn_ref[...] = jnp.sqrt(1.0 - a)

    sa = sa_ref[...].reshape(1, 1, 128)
    sn = sn_ref[...].reshape(1, 1, 128)
    n = n_ref[...]
    out_ref[...] = sa * x_ref[...] + sn * n
    nout_ref[...] = n


def kernel(x_0, noise, t, alpha_cumprod):
    B, C, H, W = x_0.shape
    T = alpha_cumprod.shape[0]
    R = C * H
    xT = jnp.transpose(x_0, (1, 2, 3, 0)).reshape(R, W, B)
    nT = jnp.transpose(noise, (1, 2, 3, 0)).reshape(R, W, B)
    alpha_p = jnp.pad(alpha_cumprod, (0, _KP - T)).reshape(1, _KP)
    t2 = t.reshape(1, B)

    blk = (_RB, W, B)
    bmap = lambda r: (r, 0, 0)
    outT, noutT = pl.pallas_call(
        _tc_body,
        grid=(R // _RB,),
        in_specs=[
            pl.BlockSpec((1, _KP), lambda r: (0, 0)),
            pl.BlockSpec((1, B), lambda r: (0, 0)),
            pl.BlockSpec(blk, bmap),
            pl.BlockSpec(blk, bmap),
        ],
        out_specs=[pl.BlockSpec(blk, bmap), pl.BlockSpec(blk, bmap)],
        out_shape=[
            jax.ShapeDtypeStruct((R, W, B), x_0.dtype),
            jax.ShapeDtypeStruct((R, W, B), x_0.dtype),
        ],
        scratch_shapes=[
            pltpu.VMEM((1, B), jnp.float32),
            pltpu.VMEM((1, B), jnp.float32),
        ],
    )(alpha_p, t2, xT, nT)
    out4 = outT.reshape(C, H, W, B)
    nout4 = noutT.reshape(C, H, W, B)
    return (
        jnp.transpose(out4, (3, 0, 1, 2)),
        jnp.transpose(nout4, (3, 0, 1, 2)),
        t,
    )
